# Initial kernel scaffold; baseline (speedup 1.0000x reference)
import functools

import jax
import jax.numpy as jnp
from jax import lax
from jax.experimental import pallas as pl
from jax.experimental.pallas import tpu as pltpu
from jax.experimental.pallas import tpu_sc as plsc

D = 64
B = 16384
N_NEG = 10
N_SIM = 1 + N_NEG
L = 16
NW = 32
BPW = B // NW
G = 128
NCHUNK = BPW // G
NG = G // L
NEG_ROWS_PER_CHUNK = G * N_NEG // G


def _sc_sims_kernel(tgt_idx_hbm, ctx_idx_hbm, neg_idx_hbm, tgt_tab, ctx_tab,
                    out_hbm,
                    tgt_idx_v, ctx_idx_v, neg_idx_v,
                    tgt_rows, ctx_rows, neg_rows, sims_v, sem):
    wid = lax.axis_index("c") * 16 + lax.axis_index("s")
    base = wid * BPW
    lane = lax.iota(jnp.int32, L)
    lane10 = lane * N_NEG

    def chunk_body(c, carry):
        cb = base + c * G
        nrow0 = wid * (BPW * N_NEG // G) + c * NEG_ROWS_PER_CHUNK
        pltpu.sync_copy(tgt_idx_hbm.at[pl.ds(cb, G)], tgt_idx_v)
        pltpu.sync_copy(ctx_idx_hbm.at[pl.ds(cb, G)], ctx_idx_v)
        pltpu.sync_copy(neg_idx_hbm.at[pl.ds(nrow0, NEG_ROWS_PER_CHUNK)],
                        neg_idx_v)
        cp_t = pltpu.async_copy(tgt_tab.at[tgt_idx_v], tgt_rows, sem)
        cp_c = pltpu.async_copy(ctx_tab.at[ctx_idx_v], ctx_rows, sem)
        cps = []
        for j in range(NEG_ROWS_PER_CHUNK):
            cps.append(pltpu.async_copy(ctx_tab.at[neg_idx_v.at[j]],
                                        neg_rows.at[pl.ds(j * G, G)], sem))
        cp_t.wait()
        cp_c.wait()
        for cp in cps:
            cp.wait()

        def group_body(g, carry2):
            e0 = g * L
            row_tc = e0 + lane
            nbase = e0 * N_NEG + lane10
            nrows = [nbase + j for j in range(N_NEG)]
            acc_c = jnp.zeros((L,), jnp.float32)
            acc_n = [jnp.zeros((L,), jnp.float32) for _ in range(N_NEG)]
            for d in range(D):
                col = jnp.full((L,), d, jnp.int32)
                t = plsc.load_gather(tgt_rows, [row_tc, col])
                cv = plsc.load_gather(ctx_rows, [row_tc, col])
                acc_c = acc_c + t * cv
                for j in range(N_NEG):
                    nv = plsc.load_gather(neg_rows, [nrows[j], col])
                    acc_n[j] = acc_n[j] + t * nv
            sims_v[0, pl.ds(e0, L)] = acc_c
            for j in range(N_NEG):
                sims_v[1 + j, pl.ds(e0, L)] = acc_n[j]
            return carry2

        lax.fori_loop(0, NG, group_body, 0)
        pltpu.sync_copy(sims_v, out_hbm.at[:, pl.ds(cb, G)])
        return carry

    lax.fori_loop(0, NCHUNK, chunk_body, 0)


@jax.jit
def _sc_sims(tgt_idx, ctx_idx, neg_idx2d, tgt_tab, ctx_tab):
    mesh = plsc.VectorSubcoreMesh(core_axis_name="c", subcore_axis_name="s")
    run = functools.partial(
        pl.kernel,
        out_type=jax.ShapeDtypeStruct((N_SIM, B), jnp.float32),
        mesh=mesh,
        scratch_types=[
            pltpu.VMEM((G,), jnp.int32),
            pltpu.VMEM((G,), jnp.int32),
            pltpu.VMEM((NEG_ROWS_PER_CHUNK, G), jnp.int32),
            pltpu.VMEM((G, D), jnp.float32),
            pltpu.VMEM((G, D), jnp.float32),
            pltpu.VMEM((G * N_NEG, D), jnp.float32),
            pltpu.VMEM((N_SIM, G), jnp.float32),
            pltpu.SemaphoreType.DMA,
        ],
    )(_sc_sims_kernel)
    return run(tgt_idx, ctx_idx, neg_idx2d, tgt_tab, ctx_tab)


def _loss_body(s_ref, o_ref):
    s = s_ref[...]
    row = lax.broadcasted_iota(jnp.int32, (N_SIM, B), 0)
    x = jnp.where(row == 0, s, -s)
    ls = jnp.minimum(x, 0.0) - jnp.log1p(jnp.exp(-jnp.abs(x)))
    o_ref[0, 0] = -jnp.sum(ls) / float(B)


@jax.jit
def _loss(sims):
    out = pl.pallas_call(
        _loss_body,
        out_shape=jax.ShapeDtypeStruct((1, 1), jnp.float32),
        out_specs=pl.BlockSpec(memory_space=pltpu.SMEM),
    )(sims)
    return out[0, 0]


def kernel(target_indices, context_indices, neg_indices, batch_size,
           B_target_w, B_context_w):
    del batch_size
    tgt_idx = target_indices.astype(jnp.int32)
    ctx_idx = context_indices.astype(jnp.int32)
    neg_idx2d = neg_indices.astype(jnp.int32).reshape(-1, G)
    sims = _sc_sims(tgt_idx, ctx_idx, neg_idx2d,
                    B_target_w, B_context_w)
    return _loss(sims)


# speedup vs baseline: 1.1721x; 1.1721x over previous
import functools

import jax
import jax.numpy as jnp
from jax import lax
from jax.experimental import pallas as pl
from jax.experimental.pallas import tpu as pltpu
from jax.experimental.pallas import tpu_sc as plsc

D = 64
B = 16384
N_NEG = 10
N_SIM = 1 + N_NEG
L = 16
NW = 32
BPW = B // NW
G = 128
NCHUNK = BPW // G
NG = G // L
NEG_ROWS_PER_CHUNK = G * N_NEG // G


def _sc_sims_kernel(tgt_idx_hbm, ctx_idx_hbm, neg_idx_hbm, tgt_tab, ctx_tab,
                    out_hbm,
                    tgt_idx_v, ctx_idx_v, neg_idx_v,
                    tgt_rows, ctx_rows, neg_rows, sims_v, sem):
    wid = lax.axis_index("c") * 16 + lax.axis_index("s")
    base = wid * BPW
    lane = lax.iota(jnp.int32, L)
    lane10 = lane * N_NEG

    def chunk_body(c, carry):
        cb = base + c * G
        pltpu.sync_copy(tgt_idx_hbm.at[pl.ds(cb, G)], tgt_idx_v)
        pltpu.sync_copy(ctx_idx_hbm.at[pl.ds(cb, G)], ctx_idx_v)
        pltpu.sync_copy(neg_idx_hbm.at[pl.ds(cb * N_NEG, G * N_NEG)],
                        neg_idx_v)
        cp_t = pltpu.async_copy(tgt_tab.at[tgt_idx_v], tgt_rows, sem)
        cp_c = pltpu.async_copy(ctx_tab.at[ctx_idx_v], ctx_rows, sem)
        cps = []
        for j in range(NEG_ROWS_PER_CHUNK):
            cps.append(
                pltpu.async_copy(ctx_tab.at[neg_idx_v.at[pl.ds(j * G, G)]],
                                 neg_rows.at[pl.ds(j * G, G)], sem))
        cp_t.wait()
        cp_c.wait()
        for cp in cps:
            cp.wait()

        def group_body(g, carry2):
            e0 = g * L
            row_tc = e0 + lane
            nbase = e0 * N_NEG + lane10
            nrows = [nbase + j for j in range(N_NEG)]
            acc_c = jnp.zeros((L,), jnp.float32)
            acc_n = [jnp.zeros((L,), jnp.float32) for _ in range(N_NEG)]
            for d in range(D):
                col = jnp.full((L,), d, jnp.int32)
                t = plsc.load_gather(tgt_rows, [row_tc, col])
                cv = plsc.load_gather(ctx_rows, [row_tc, col])
                acc_c = acc_c + t * cv
                for j in range(N_NEG):
                    nv = plsc.load_gather(neg_rows, [nrows[j], col])
                    acc_n[j] = acc_n[j] + t * nv
            sims_v[0, pl.ds(e0, L)] = acc_c
            for j in range(N_NEG):
                sims_v[1 + j, pl.ds(e0, L)] = acc_n[j]
            return carry2

        lax.fori_loop(0, NG, group_body, 0)
        pltpu.sync_copy(sims_v, out_hbm.at[:, pl.ds(cb, G)])
        return carry

    lax.fori_loop(0, NCHUNK, chunk_body, 0)


@jax.jit
def _sc_sims(tgt_idx, ctx_idx, neg_idx_flat, tgt_tab, ctx_tab):
    mesh = plsc.VectorSubcoreMesh(core_axis_name="c", subcore_axis_name="s")
    run = functools.partial(
        pl.kernel,
        out_type=jax.ShapeDtypeStruct((N_SIM, B), jnp.float32),
        mesh=mesh,
        compiler_params=pltpu.CompilerParams(needs_layout_passes=False,
                                             use_tc_tiling_on_sc=False),
        scratch_types=[
            pltpu.VMEM((G,), jnp.int32),
            pltpu.VMEM((G,), jnp.int32),
            pltpu.VMEM((G * N_NEG,), jnp.int32),
            pltpu.VMEM((G, D), jnp.float32),
            pltpu.VMEM((G, D), jnp.float32),
            pltpu.VMEM((G * N_NEG, D), jnp.float32),
            pltpu.VMEM((N_SIM, G), jnp.float32),
            pltpu.SemaphoreType.DMA,
        ],
    )(_sc_sims_kernel)
    return run(tgt_idx, ctx_idx, neg_idx_flat, tgt_tab, ctx_tab)


def _loss_body(s_ref, o_ref):
    s = s_ref[...]
    row = lax.broadcasted_iota(jnp.int32, (N_SIM, B), 0)
    x = jnp.where(row == 0, s, -s)
    ls = jnp.minimum(x, 0.0) - jnp.log1p(jnp.exp(-jnp.abs(x)))
    o_ref[0, 0] = -jnp.sum(ls) / float(B)


@jax.jit
def _loss(sims):
    out = pl.pallas_call(
        _loss_body,
        out_shape=jax.ShapeDtypeStruct((1, 1), jnp.float32),
        out_specs=pl.BlockSpec(memory_space=pltpu.SMEM),
    )(sims)
    return out[0, 0]


def kernel(target_indices, context_indices, neg_indices, batch_size,
           B_target_w, B_context_w):
    del batch_size
    tgt_idx = target_indices.astype(jnp.int32)
    ctx_idx = context_indices.astype(jnp.int32)
    neg_idx_flat = neg_indices.astype(jnp.int32).reshape(-1)
    sims = _sc_sims(tgt_idx, ctx_idx, neg_idx_flat,
                    B_target_w, B_context_w)
    return _loss(sims)
